# baseline (speedup 1.0000x reference)
import jax
import jax.numpy as jnp
from jax import lax
from jax.experimental import pallas as pl
from jax.experimental.pallas import tpu as pltpu
from jax.experimental.pallas import tpu_sc as plsc

N = 10000
E = 320000
H = 128
HEADS = 8
HD = H // HEADS

NC = 2
NS = 16
NW = NC * NS
EW = E // NW
C = 40
NCHUNK = EW // C
NP = 10240
RPS = NP // NS
NB = 10
BN = N // NB


def _qkv_body(x_ref, wq_ref, wk_ref, wv_ref, bq_ref, bk_ref, bv_ref,
              q_ref, kv_ref):
    xb = x_ref[...]
    q = jnp.dot(xb, wq_ref[...], preferred_element_type=jnp.float32) + bq_ref[...]
    k = jnp.dot(xb, wk_ref[...], preferred_element_type=jnp.float32) + bk_ref[...]
    v = jnp.dot(xb, wv_ref[...], preferred_element_type=jnp.float32) + bv_ref[...]
    q_ref[...] = q
    kv_ref[...] = jnp.concatenate([k, v], axis=1)


def _qkv_call(x, Wq, bq, Wk, bk, Wv, bv):
    f32 = jnp.float32
    return pl.pallas_call(
        _qkv_body,
        grid=(NB,),
        in_specs=[
            pl.BlockSpec((BN, H), lambda i: (i, 0)),
            pl.BlockSpec((H, H), lambda i: (0, 0)),
            pl.BlockSpec((H, H), lambda i: (0, 0)),
            pl.BlockSpec((H, H), lambda i: (0, 0)),
            pl.BlockSpec((1, H), lambda i: (0, 0)),
            pl.BlockSpec((1, H), lambda i: (0, 0)),
            pl.BlockSpec((1, H), lambda i: (0, 0)),
        ],
        out_specs=[
            pl.BlockSpec((BN, H), lambda i: (i, 0)),
            pl.BlockSpec((BN, 2 * H), lambda i: (i, 0)),
        ],
        out_shape=[
            jax.ShapeDtypeStruct((N, H), f32),
            jax.ShapeDtypeStruct((N, 2 * H), f32),
        ],
    )(x, Wq.T, Wk.T, Wv.T, bq.reshape(1, H), bk.reshape(1, H),
      bv.reshape(1, H))


def _edge_body(q_hbm, kv_hbm, src_hbm, dst_hbm, p_out, z_out,
               srcidx0, dstidx0, qe0, kve0, srcidx1, dstidx1, qe1, kve1,
               uref, sref, zvmem, shared, semq0, semk0, semq1, semk1):
    c = lax.axis_index("c")
    s = lax.axis_index("s")
    wid = s * NC + c

    srcidx = (srcidx0, srcidx1)
    dstidx = (dstidx0, dstidx1)
    qe = (qe0, qe1)
    kve = (kve0, kve1)
    semq = (semq0, semq1)
    semk = (semk0, semk1)

    def _zrow_body(r, carry):
        for j in range(H // 16):
            qe0[r, pl.ds(j * 16, 16)] = jnp.zeros((16,), jnp.float32)
        return carry
    lax.fori_loop(0, C, _zrow_body, 0)
    for j in range(RPS // C):
        pltpu.sync_copy(qe0, shared.at[pl.ds(s * RPS + j * C, C)])
    plsc.subcore_barrier()

    lane = lax.iota(jnp.int32, 16)
    mask15 = lane == 15

    def _issue(b, ci):
        base = wid * EW + ci * C
        pltpu.sync_copy(src_hbm.at[pl.ds(base, C)], srcidx[b])
        pltpu.sync_copy(dst_hbm.at[pl.ds(base, C)], dstidx[b])
        pltpu.async_copy(q_hbm.at[dstidx[b]], qe[b], semq[b])
        pltpu.async_copy(kv_hbm.at[srcidx[b]], kve[b], semk[b])

    def _wait(b):
        pltpu.make_async_copy(q_hbm.at[dstidx[b]], qe[b], semq[b]).wait()
        pltpu.make_async_copy(kv_hbm.at[srcidx[b]], kve[b], semk[b]).wait()

    def _compute(b, zacc):
        qe_b, kve_b = qe[b], kve[b]

        def _score_body(g, zin):
            sbase = g * 16
            for e in range(2):
                i = 2 * g + e
                for h in range(HEADS):
                    qv = qe_b[i, pl.ds(h * HD, HD)]
                    kv = kve_b[i, pl.ds(h * HD, HD)]
                    sc = plsc.cumsum(qv * kv)
                    plsc.store_scatter(
                        sref, [jnp.full((16,), 0, jnp.int32) + (sbase + e * 8 + h)],
                        sc, mask=mask15)
            u2 = jnp.exp(sref[pl.ds(sbase, 16)] * 0.25)
            uref[g, :] = u2
            return zin + u2
        zacc = lax.fori_loop(0, C // 2, _score_body, zacc)

        def _scale_body(g, carry):
            gfull = jnp.full((16,), 0, jnp.int32) + g
            for par in range(2):
                i = 2 * g + par
                for h in range(HEADS):
                    wsplat = plsc.load_gather(
                        uref, [gfull, jnp.full((16,), par * 8 + h, jnp.int32)])
                    vvec = kve_b[i, pl.ds(H + h * HD, HD)]
                    qe_b[i, pl.ds(h * HD, HD)] = vvec * wsplat
            return carry
        lax.fori_loop(0, C // 2, _scale_body, 0)

        pltpu.sync_copy(qe_b, shared.at[dstidx[b]], add=True)
        return zacc

    _issue(0, 0)

    def _super_body(mc, zacc):
        a = 2 * mc
        _issue(1, a + 1)
        _wait(0)
        zacc = _compute(0, zacc)
        _issue(0, lax.rem(a + 2, NCHUNK))
        _wait(1)
        zacc = _compute(1, zacc)
        return zacc

    zacc = lax.fori_loop(0, NCHUNK // 2, _super_body,
                         jnp.zeros((16,), jnp.float32))
    _wait(0)

    zvmem[...] = zacc
    pltpu.sync_copy(zvmem, z_out.at[wid])

    plsc.subcore_barrier()
    pltpu.sync_copy(shared.at[pl.ds(s * RPS, RPS)],
                    p_out.at[c, pl.ds(s * RPS, RPS)])


def _edge_call(q, kv, src, dst):
    f32 = jnp.float32
    mesh = plsc.VectorSubcoreMesh(core_axis_name="c", subcore_axis_name="s",
                                  num_cores=NC, num_subcores=NS)
    return pl.kernel(
        _edge_body,
        out_type=[
            jax.ShapeDtypeStruct((NC, NP, H), f32),
            jax.ShapeDtypeStruct((NW, 16), f32),
        ],
        mesh=mesh,
        compiler_params=pltpu.CompilerParams(needs_layout_passes=False),
        scratch_types=[
            pltpu.VMEM((C,), jnp.int32),
            pltpu.VMEM((C,), jnp.int32),
            pltpu.VMEM((C, H), f32),
            pltpu.VMEM((C, 2 * H), f32),
            pltpu.VMEM((C,), jnp.int32),
            pltpu.VMEM((C,), jnp.int32),
            pltpu.VMEM((C, H), f32),
            pltpu.VMEM((C, 2 * H), f32),
            pltpu.VMEM((C // 2, 16), f32),
            pltpu.VMEM((C * 8,), f32),
            pltpu.VMEM((16,), f32),
            pltpu.VMEM_SHARED((NP, H), f32),
            pltpu.SemaphoreType.DMA,
            pltpu.SemaphoreType.DMA,
            pltpu.SemaphoreType.DMA,
            pltpu.SemaphoreType.DMA,
        ],
    )(q, kv, src, dst)


def _out_body(x_ref, p_ref, zp_ref, wo_ref, bo_ref, o_ref):
    zp = zp_ref[...]
    ones = jnp.ones((1, NW), jnp.float32)
    z16 = jnp.dot(ones, zp, preferred_element_type=jnp.float32)
    z8 = z16[:, :8] + z16[:, 8:]
    invz = 1.0 / z8
    rowi = lax.broadcasted_iota(jnp.int32, (HEADS, H), 0)
    coli = lax.broadcasted_iota(jnp.int32, (HEADS, H), 1)
    expand = jnp.where(coli // HD == rowi, 1.0, 0.0).astype(jnp.float32)
    scale = jnp.dot(invz, expand, preferred_element_type=jnp.float32)
    att = (p_ref[0] + p_ref[1]) * scale
    o_ref[...] = (x_ref[...]
                  + jnp.dot(att, wo_ref[...], preferred_element_type=jnp.float32)
                  + bo_ref[...])


def _out_call(x, p, zp, Wo, bo):
    f32 = jnp.float32
    return pl.pallas_call(
        _out_body,
        grid=(NB,),
        in_specs=[
            pl.BlockSpec((BN, H), lambda i: (i, 0)),
            pl.BlockSpec((NC, BN, H), lambda i: (0, i, 0)),
            pl.BlockSpec((NW, 16), lambda i: (0, 0)),
            pl.BlockSpec((H, H), lambda i: (0, 0)),
            pl.BlockSpec((1, H), lambda i: (0, 0)),
        ],
        out_specs=pl.BlockSpec((BN, H), lambda i: (i, 0)),
        out_shape=jax.ShapeDtypeStruct((N, H), f32),
    )(x, p, zp, Wo.T, bo.reshape(1, H))


def kernel(x, gap_edge_index, Wq, bq, Wk, bk, Wv, bv, Wo, bo):
    src = gap_edge_index[0]
    dst = gap_edge_index[1]
    q, kv = _qkv_call(x, Wq, bq, Wk, bk, Wv, bv)
    p, zp = _edge_call(q, kv, src, dst)
    return _out_call(x, p, zp, Wo, bo)


# speedup vs baseline: 1.5828x; 1.5828x over previous
import jax
import jax.numpy as jnp
from jax import lax
from jax.experimental import pallas as pl
from jax.experimental.pallas import tpu as pltpu
from jax.experimental.pallas import tpu_sc as plsc

N = 10000
E = 320000
H = 128
HEADS = 8
HD = H // HEADS

NC = 2
NS = 16
NW = NC * NS
EW = E // NW
C = 40
NCHUNK = EW // C
NP = 10240
RPS = NP // NS
NB = 10
BN = N // NB


def _qkv_body(x_ref, wq_ref, wk_ref, wv_ref, bq_ref, bk_ref, bv_ref,
              q_ref, kv_ref):
    xb = x_ref[...]
    q = jnp.dot(xb, wq_ref[...], preferred_element_type=jnp.float32) + bq_ref[...]
    k = jnp.dot(xb, wk_ref[...], preferred_element_type=jnp.float32) + bk_ref[...]
    v = jnp.dot(xb, wv_ref[...], preferred_element_type=jnp.float32) + bv_ref[...]
    q_ref[...] = q
    kv_ref[...] = jnp.concatenate([k, v], axis=1)


def _qkv_call(x, Wq, bq, Wk, bk, Wv, bv):
    f32 = jnp.float32
    return pl.pallas_call(
        _qkv_body,
        grid=(NB,),
        in_specs=[
            pl.BlockSpec((BN, H), lambda i: (i, 0)),
            pl.BlockSpec((H, H), lambda i: (0, 0)),
            pl.BlockSpec((H, H), lambda i: (0, 0)),
            pl.BlockSpec((H, H), lambda i: (0, 0)),
            pl.BlockSpec((1, H), lambda i: (0, 0)),
            pl.BlockSpec((1, H), lambda i: (0, 0)),
            pl.BlockSpec((1, H), lambda i: (0, 0)),
        ],
        out_specs=[
            pl.BlockSpec((BN, H), lambda i: (i, 0)),
            pl.BlockSpec((BN, 2 * H), lambda i: (i, 0)),
        ],
        out_shape=[
            jax.ShapeDtypeStruct((N, H), f32),
            jax.ShapeDtypeStruct((N, 2 * H), f32),
        ],
    )(x, Wq.T, Wk.T, Wv.T, bq.reshape(1, H), bk.reshape(1, H),
      bv.reshape(1, H))


def _edge_body(q_hbm, kv_hbm, src_hbm, dst_hbm, p_out, z_out,
               srcidx0, dstidx0, qe0, kve0, srcidx1, dstidx1, qe1, kve1,
               uref, zvmem, shared, dsts0, dsts1, semq0, semk0, semq1, semk1,
               semi0, semi1, semd0, semd1):
    c = lax.axis_index("c")
    s = lax.axis_index("s")
    wid = s * NC + c

    srcidx = (srcidx0, srcidx1)
    dstidx = (dstidx0, dstidx1)
    qe = (qe0, qe1)
    kve = (kve0, kve1)
    semq = (semq0, semq1)
    semk = (semk0, semk1)
    semi = (semi0, semi1)
    dsts = (dsts0, dsts1)
    semd = (semd0, semd1)

    def _zrow_body(r, carry):
        for j in range(H // 16):
            qe0[r, pl.ds(j * 16, 16)] = jnp.zeros((16,), jnp.float32)
        return carry
    lax.fori_loop(0, C, _zrow_body, 0)
    for j in range(RPS // C):
        pltpu.sync_copy(qe0, shared.at[pl.ds(s * RPS + j * C, C)])
    plsc.subcore_barrier()

    lane = lax.iota(jnp.int32, 16)
    halfsel = lax.shift_right_logical(lane, 3)
    colbase = lax.bitwise_and(lane, 7) * HD

    def _issue_idx(b, ci):
        base = wid * EW + ci * C
        pltpu.async_copy(src_hbm.at[pl.ds(base, C)], srcidx[b], semi[b])
        pltpu.async_copy(dst_hbm.at[pl.ds(base, C)], dstidx[b], semi[b])

    def _wait_idx(b, ci):
        base = wid * EW + ci * C
        pltpu.make_async_copy(src_hbm.at[pl.ds(base, C)], srcidx[b], semi[b]).wait()
        pltpu.make_async_copy(dst_hbm.at[pl.ds(base, C)], dstidx[b], semi[b]).wait()

    def _issue_rows(b, ci):
        base = wid * EW + ci * C
        pltpu.async_copy(q_hbm.at[dstidx[b]], qe[b], semq[b])
        pltpu.async_copy(kv_hbm.at[srcidx[b]], kve[b], semk[b])
        pltpu.async_copy(dst_hbm.at[pl.ds(base, C)], dsts[b], semd[b])

    def _wait_rows(b, ci):
        base = wid * EW + ci * C
        pltpu.make_async_copy(q_hbm.at[dstidx[b]], qe[b], semq[b]).wait()
        pltpu.make_async_copy(kv_hbm.at[srcidx[b]], kve[b], semk[b]).wait()
        pltpu.make_async_copy(dst_hbm.at[pl.ds(base, C)], dsts[b], semd[b]).wait()

    def _compute(b, zacc):
        qe_b, kve_b = qe[b], kve[b]

        def _score_body(g, zin):
            rowv = 2 * g + halfsel
            acc = jnp.zeros((16,), jnp.float32)
            for d in range(HD):
                colv = colbase + d
                qv = plsc.load_gather(qe_b, [rowv, colv])
                kv = plsc.load_gather(kve_b, [rowv, colv])
                acc = acc + qv * kv
            u2 = jnp.exp(acc * 0.25)
            uref[g, :] = u2
            return zin + u2
        zacc = lax.fori_loop(0, C // 2, _score_body, zacc)

        def _scale_body(g, carry):
            uvec = uref[g, :]
            for par in range(2):
                i = 2 * g + par
                for h in range(HEADS):
                    w = uvec[par * 8 + h]
                    vvec = kve_b[i, pl.ds(H + h * HD, HD)]
                    qe_b[i, pl.ds(h * HD, HD)] = vvec * w
            return carry
        lax.fori_loop(0, C // 2, _scale_body, 0)

        pltpu.sync_copy(qe_b, shared.at[dsts[b]], add=True)
        return zacc

    _issue_idx(0, 0)
    _wait_idx(0, 0)
    _issue_rows(0, 0)
    _issue_idx(1, 1)

    def _super_body(mc, zacc):
        a = 2 * mc
        a2 = lax.rem(a + 2, NCHUNK)
        _wait_idx(1, a + 1)
        _issue_rows(1, a + 1)
        _wait_rows(0, a)
        _issue_idx(0, a2)
        zacc = _compute(0, zacc)
        _wait_idx(0, a2)
        _issue_rows(0, a2)
        _wait_rows(1, a + 1)
        _issue_idx(1, lax.rem(a + 3, NCHUNK))
        zacc = _compute(1, zacc)
        return zacc

    zacc = lax.fori_loop(0, NCHUNK // 2, _super_body,
                         jnp.zeros((16,), jnp.float32))
    _wait_rows(0, 0)
    _wait_idx(1, 1)

    zvmem[...] = zacc
    pltpu.sync_copy(zvmem, z_out.at[wid])

    plsc.subcore_barrier()
    pltpu.sync_copy(shared.at[pl.ds(s * RPS, RPS)],
                    p_out.at[c, pl.ds(s * RPS, RPS)])


def _edge_call(q, kv, src, dst):
    f32 = jnp.float32
    mesh = plsc.VectorSubcoreMesh(core_axis_name="c", subcore_axis_name="s",
                                  num_cores=NC, num_subcores=NS)
    return pl.kernel(
        _edge_body,
        out_type=[
            jax.ShapeDtypeStruct((NC, NP, H), f32),
            jax.ShapeDtypeStruct((NW, 16), f32),
        ],
        mesh=mesh,
        compiler_params=pltpu.CompilerParams(needs_layout_passes=False),
        scratch_types=[
            pltpu.VMEM((C,), jnp.int32),
            pltpu.VMEM((C,), jnp.int32),
            pltpu.VMEM((C, H), f32),
            pltpu.VMEM((C, 2 * H), f32),
            pltpu.VMEM((C,), jnp.int32),
            pltpu.VMEM((C,), jnp.int32),
            pltpu.VMEM((C, H), f32),
            pltpu.VMEM((C, 2 * H), f32),
            pltpu.VMEM((C // 2, 16), f32),
            pltpu.VMEM((16,), f32),
            pltpu.VMEM_SHARED((NP, H), f32),
            pltpu.VMEM((C,), jnp.int32),
            pltpu.VMEM((C,), jnp.int32),
            pltpu.SemaphoreType.DMA,
            pltpu.SemaphoreType.DMA,
            pltpu.SemaphoreType.DMA,
            pltpu.SemaphoreType.DMA,
            pltpu.SemaphoreType.DMA,
            pltpu.SemaphoreType.DMA,
            pltpu.SemaphoreType.DMA,
            pltpu.SemaphoreType.DMA,
        ],
    )(q, kv, src, dst)


def _out_body(x_ref, p_ref, zp_ref, wo_ref, bo_ref, o_ref):
    zp = zp_ref[...]
    ones = jnp.ones((1, NW), jnp.float32)
    z16 = jnp.dot(ones, zp, preferred_element_type=jnp.float32)
    z8 = z16[:, :8] + z16[:, 8:]
    invz = 1.0 / z8
    rowi = lax.broadcasted_iota(jnp.int32, (HEADS, H), 0)
    coli = lax.broadcasted_iota(jnp.int32, (HEADS, H), 1)
    expand = jnp.where(coli // HD == rowi, 1.0, 0.0).astype(jnp.float32)
    scale = jnp.dot(invz, expand, preferred_element_type=jnp.float32)
    att = (p_ref[0] + p_ref[1]) * scale
    o_ref[...] = (x_ref[...]
                  + jnp.dot(att, wo_ref[...], preferred_element_type=jnp.float32)
                  + bo_ref[...])


def _out_call(x, p, zp, Wo, bo):
    f32 = jnp.float32
    return pl.pallas_call(
        _out_body,
        grid=(NB,),
        in_specs=[
            pl.BlockSpec((BN, H), lambda i: (i, 0)),
            pl.BlockSpec((NC, BN, H), lambda i: (0, i, 0)),
            pl.BlockSpec((NW, 16), lambda i: (0, 0)),
            pl.BlockSpec((H, H), lambda i: (0, 0)),
            pl.BlockSpec((1, H), lambda i: (0, 0)),
        ],
        out_specs=pl.BlockSpec((BN, H), lambda i: (i, 0)),
        out_shape=jax.ShapeDtypeStruct((N, H), f32),
    )(x, p, zp, Wo.T, bo.reshape(1, H))


def kernel(x, gap_edge_index, Wq, bq, Wk, bk, Wv, bv, Wo, bo):
    src = gap_edge_index[0]
    dst = gap_edge_index[1]
    q, kv = _qkv_call(x, Wq, bq, Wk, bk, Wv, bv)
    p, zp = _edge_call(q, kv, src, dst)
    return _out_call(x, p, zp, Wo, bo)


# speedup vs baseline: 1.5963x; 1.0085x over previous
import jax
import jax.numpy as jnp
from jax import lax
from jax.experimental import pallas as pl
from jax.experimental.pallas import tpu as pltpu
from jax.experimental.pallas import tpu_sc as plsc

N = 10000
E = 320000
H = 128
HEADS = 8
HD = H // HEADS

NC = 2
NS = 16
NW = NC * NS
EW = E // NW
C = 40
NCHUNK = EW // C
NP = 10240
RPS = NP // NS
NB = 10
BN = N // NB


def _qkv_body(x_ref, wq_ref, wk_ref, wv_ref, bq_ref, bk_ref, bv_ref,
              q_ref, kv_ref):
    xb = x_ref[...]
    q = jnp.dot(xb, wq_ref[...], preferred_element_type=jnp.float32) + bq_ref[...]
    k = jnp.dot(xb, wk_ref[...], preferred_element_type=jnp.float32) + bk_ref[...]
    v = jnp.dot(xb, wv_ref[...], preferred_element_type=jnp.float32) + bv_ref[...]
    q_ref[...] = q
    kv_ref[...] = jnp.concatenate([k, v], axis=1)


def _qkv_call(x, Wq, bq, Wk, bk, Wv, bv):
    f32 = jnp.float32
    return pl.pallas_call(
        _qkv_body,
        grid=(NB,),
        in_specs=[
            pl.BlockSpec((BN, H), lambda i: (i, 0)),
            pl.BlockSpec((H, H), lambda i: (0, 0)),
            pl.BlockSpec((H, H), lambda i: (0, 0)),
            pl.BlockSpec((H, H), lambda i: (0, 0)),
            pl.BlockSpec((1, H), lambda i: (0, 0)),
            pl.BlockSpec((1, H), lambda i: (0, 0)),
            pl.BlockSpec((1, H), lambda i: (0, 0)),
        ],
        out_specs=[
            pl.BlockSpec((BN, H), lambda i: (i, 0)),
            pl.BlockSpec((BN, 2 * H), lambda i: (i, 0)),
        ],
        out_shape=[
            jax.ShapeDtypeStruct((N, H), f32),
            jax.ShapeDtypeStruct((N, 2 * H), f32),
        ],
    )(x, Wq.T, Wk.T, Wv.T, bq.reshape(1, H), bk.reshape(1, H),
      bv.reshape(1, H))


def _edge_body(q_hbm, kv_hbm, src_hbm, dst_hbm, p_out, z_out,
               srcidx0, dstidx0, qe0, kve0, srcidx1, dstidx1, qe1, kve1,
               uref, zvmem, shared, dsts0, dsts1, wv0, wv1,
               semq0, semk0, semq1, semk1,
               semi0, semi1, semd0, semd1, semw0, semw1):
    c = lax.axis_index("c")
    s = lax.axis_index("s")
    wid = s * NC + c

    srcidx = (srcidx0, srcidx1)
    dstidx = (dstidx0, dstidx1)
    qe = (qe0, qe1)
    kve = (kve0, kve1)
    semq = (semq0, semq1)
    semk = (semk0, semk1)
    semi = (semi0, semi1)
    dsts = (dsts0, dsts1)
    semd = (semd0, semd1)
    wv = (wv0, wv1)
    semw = (semw0, semw1)

    def _zrow_body(r, carry):
        for j in range(H // 16):
            qe0[r, pl.ds(j * 16, 16)] = jnp.zeros((16,), jnp.float32)
        return carry
    lax.fori_loop(0, C, _zrow_body, 0)
    for j in range(RPS // C):
        pltpu.sync_copy(qe0, shared.at[pl.ds(s * RPS + j * C, C)])
    plsc.subcore_barrier()

    lane = lax.iota(jnp.int32, 16)
    halfsel = lax.shift_right_logical(lane, 3)
    colbase = lax.bitwise_and(lane, 7) * HD

    def _issue_idx(b, ci):
        base = wid * EW + ci * C
        pltpu.async_copy(src_hbm.at[pl.ds(base, C)], srcidx[b], semi[b])
        pltpu.async_copy(dst_hbm.at[pl.ds(base, C)], dstidx[b], semi[b])

    def _wait_idx(b, ci):
        base = wid * EW + ci * C
        pltpu.make_async_copy(src_hbm.at[pl.ds(base, C)], srcidx[b], semi[b]).wait()
        pltpu.make_async_copy(dst_hbm.at[pl.ds(base, C)], dstidx[b], semi[b]).wait()

    def _issue_rows(b, ci, first=jnp.bool_(False)):
        base = wid * EW + ci * C
        pltpu.async_copy(q_hbm.at[dstidx[b]], qe[b], semq[b])
        pltpu.async_copy(kv_hbm.at[srcidx[b]], kve[b], semk[b])
        @pl.when(jnp.logical_not(first))
        def _():
            _scatter_wait(b)
        pltpu.async_copy(dst_hbm.at[pl.ds(base, C)], dsts[b], semd[b])

    def _wait_rows(b, ci):
        base = wid * EW + ci * C
        pltpu.make_async_copy(q_hbm.at[dstidx[b]], qe[b], semq[b]).wait()
        pltpu.make_async_copy(kv_hbm.at[srcidx[b]], kve[b], semk[b]).wait()
        pltpu.make_async_copy(dst_hbm.at[pl.ds(base, C)], dsts[b], semd[b]).wait()

    def _scatter_wait(b):
        pltpu.make_async_copy(wv[b], shared.at[dsts[b]], semw[b]).wait()

    def _compute(b, zacc):
        qe_b, kve_b, wv_b = qe[b], kve[b], wv[b]

        def _score_body(g, zin):
            rowv = 2 * g + halfsel
            acc = jnp.zeros((16,), jnp.float32)
            for d in range(HD):
                colv = colbase + d
                qv = plsc.load_gather(qe_b, [rowv, colv])
                kv = plsc.load_gather(kve_b, [rowv, colv])
                acc = acc + qv * kv
            u2 = jnp.exp(acc * 0.25)
            uref[g, :] = u2
            return zin + u2
        zacc = lax.fori_loop(0, C // 2, _score_body, zacc)

        def _scale_body(g, carry):
            uvec = uref[g, :]
            for par in range(2):
                i = 2 * g + par
                for h in range(HEADS):
                    w = uvec[par * 8 + h]
                    vvec = kve_b[i, pl.ds(H + h * HD, HD)]
                    wv_b[i, pl.ds(h * HD, HD)] = vvec * w
            return carry
        lax.fori_loop(0, C // 2, _scale_body, 0)

        pltpu.make_async_copy(wv_b, shared.at[dsts[b]], semw[b]).start(add=True)
        return zacc

    _issue_idx(0, 0)
    _wait_idx(0, 0)
    _issue_rows(0, 0, first=jnp.bool_(True))
    _issue_idx(1, 1)

    def _super_body(mc, zacc):
        a = 2 * mc
        a2 = lax.rem(a + 2, NCHUNK)
        _wait_idx(1, a + 1)
        _issue_rows(1, a + 1, first=(mc == 0))
        _wait_rows(0, a)
        _issue_idx(0, a2)
        zacc = _compute(0, zacc)
        _wait_idx(0, a2)
        _issue_rows(0, a2)
        _wait_rows(1, a + 1)
        _issue_idx(1, lax.rem(a + 3, NCHUNK))
        zacc = _compute(1, zacc)
        return zacc

    zacc = lax.fori_loop(0, NCHUNK // 2, _super_body,
                         jnp.zeros((16,), jnp.float32))
    _wait_rows(0, 0)
    _wait_idx(1, 1)
    _scatter_wait(1)

    zvmem[...] = zacc
    pltpu.sync_copy(zvmem, z_out.at[wid])

    plsc.subcore_barrier()
    pltpu.sync_copy(shared.at[pl.ds(s * RPS, RPS)],
                    p_out.at[c, pl.ds(s * RPS, RPS)])


def _edge_call(q, kv, src, dst):
    f32 = jnp.float32
    mesh = plsc.VectorSubcoreMesh(core_axis_name="c", subcore_axis_name="s",
                                  num_cores=NC, num_subcores=NS)
    return pl.kernel(
        _edge_body,
        out_type=[
            jax.ShapeDtypeStruct((NC, NP, H), f32),
            jax.ShapeDtypeStruct((NW, 16), f32),
        ],
        mesh=mesh,
        compiler_params=pltpu.CompilerParams(needs_layout_passes=False),
        scratch_types=[
            pltpu.VMEM((C,), jnp.int32),
            pltpu.VMEM((C,), jnp.int32),
            pltpu.VMEM((C, H), f32),
            pltpu.VMEM((C, 2 * H), f32),
            pltpu.VMEM((C,), jnp.int32),
            pltpu.VMEM((C,), jnp.int32),
            pltpu.VMEM((C, H), f32),
            pltpu.VMEM((C, 2 * H), f32),
            pltpu.VMEM((C // 2, 16), f32),
            pltpu.VMEM((16,), f32),
            pltpu.VMEM_SHARED((NP, H), f32),
            pltpu.VMEM((C,), jnp.int32),
            pltpu.VMEM((C,), jnp.int32),
            pltpu.VMEM((C, H), f32),
            pltpu.VMEM((C, H), f32),
            pltpu.SemaphoreType.DMA,
            pltpu.SemaphoreType.DMA,
            pltpu.SemaphoreType.DMA,
            pltpu.SemaphoreType.DMA,
            pltpu.SemaphoreType.DMA,
            pltpu.SemaphoreType.DMA,
            pltpu.SemaphoreType.DMA,
            pltpu.SemaphoreType.DMA,
            pltpu.SemaphoreType.DMA,
            pltpu.SemaphoreType.DMA,
        ],
    )(q, kv, src, dst)


def _out_body(x_ref, p_ref, zp_ref, wo_ref, bo_ref, o_ref):
    zp = zp_ref[...]
    ones = jnp.ones((1, NW), jnp.float32)
    z16 = jnp.dot(ones, zp, preferred_element_type=jnp.float32)
    z8 = z16[:, :8] + z16[:, 8:]
    invz = 1.0 / z8
    rowi = lax.broadcasted_iota(jnp.int32, (HEADS, H), 0)
    coli = lax.broadcasted_iota(jnp.int32, (HEADS, H), 1)
    expand = jnp.where(coli // HD == rowi, 1.0, 0.0).astype(jnp.float32)
    scale = jnp.dot(invz, expand, preferred_element_type=jnp.float32)
    att = (p_ref[0] + p_ref[1]) * scale
    o_ref[...] = (x_ref[...]
                  + jnp.dot(att, wo_ref[...], preferred_element_type=jnp.float32)
                  + bo_ref[...])


def _out_call(x, p, zp, Wo, bo):
    f32 = jnp.float32
    return pl.pallas_call(
        _out_body,
        grid=(NB,),
        in_specs=[
            pl.BlockSpec((BN, H), lambda i: (i, 0)),
            pl.BlockSpec((NC, BN, H), lambda i: (0, i, 0)),
            pl.BlockSpec((NW, 16), lambda i: (0, 0)),
            pl.BlockSpec((H, H), lambda i: (0, 0)),
            pl.BlockSpec((1, H), lambda i: (0, 0)),
        ],
        out_specs=pl.BlockSpec((BN, H), lambda i: (i, 0)),
        out_shape=jax.ShapeDtypeStruct((N, H), f32),
    )(x, p, zp, Wo.T, bo.reshape(1, H))


def kernel(x, gap_edge_index, Wq, bq, Wk, bk, Wv, bv, Wo, bo):
    src = gap_edge_index[0]
    dst = gap_edge_index[1]
    q, kv = _qkv_call(x, Wq, bq, Wk, bk, Wv, bv)
    p, zp = _edge_call(q, kv, src, dst)
    return _out_call(x, p, zp, Wo, bo)
